# Initial kernel scaffold; baseline (speedup 1.0000x reference)
import jax
import jax.numpy as jnp
from jax.experimental import pallas as pl


def kernel(input, w):
    raise NotImplementedError("write your pallas kernel here")


# speedup vs baseline: 2.1125x; 2.1125x over previous
import jax
import jax.numpy as jnp
from jax.experimental import pallas as pl
from jax.experimental.pallas import tpu as pltpu

DIM = 256
N_EMBED = 8192
COMMITMENT_COST = 0.25
TM = 256


def _vq_body(x_ref, w_ref, loss_ref, perp_ref, w2_ref, counts_ref, acc_ref):
    i = pl.program_id(0)
    ni = pl.num_programs(0)

    @pl.when(i == 0)
    def _init():
        w = w_ref[...]
        w2_ref[...] = jnp.sum(w * w, axis=0, keepdims=True)
        counts_ref[...] = jnp.zeros_like(counts_ref)
        acc_ref[...] = jnp.zeros_like(acc_ref)

    x = x_ref[...]
    scores = jnp.dot(x, w_ref[...], preferred_element_type=jnp.float32)
    scores = w2_ref[...] - 2.0 * scores
    m = jnp.min(scores, axis=1, keepdims=True)
    x2 = jnp.sum(x * x, axis=1, keepdims=True)
    acc_ref[...] += jnp.reshape(jnp.sum(m) + jnp.sum(x2), (1, 1))
    onehot = jnp.where(scores == m, 1.0, 0.0)
    counts_ref[...] += jnp.sum(onehot, axis=0, keepdims=True)

    @pl.when(i == ni - 1)
    def _fin():
        total = jnp.float32(TM) * jnp.float32(ni)
        p = counts_ref[...] / total
        ent = jnp.sum(p * jnp.log(p + 1e-10))
        perp_ref[...] = jnp.reshape(jnp.exp(-ent), (1, 1))
        loss_ref[...] = COMMITMENT_COST * acc_ref[...] / (total * DIM)


def kernel(input, w):
    x = input.reshape(-1, DIM)
    n = x.shape[0]
    ni = n // TM
    loss, perp = pl.pallas_call(
        _vq_body,
        grid=(ni,),
        in_specs=[
            pl.BlockSpec((TM, DIM), lambda i: (i, 0)),
            pl.BlockSpec((DIM, N_EMBED), lambda i: (0, 0)),
        ],
        out_specs=[
            pl.BlockSpec((1, 1), lambda i: (0, 0)),
            pl.BlockSpec((1, 1), lambda i: (0, 0)),
        ],
        out_shape=[
            jax.ShapeDtypeStruct((1, 1), jnp.float32),
            jax.ShapeDtypeStruct((1, 1), jnp.float32),
        ],
        scratch_shapes=[
            pltpu.VMEM((1, N_EMBED), jnp.float32),
            pltpu.VMEM((1, N_EMBED), jnp.float32),
            pltpu.VMEM((1, 1), jnp.float32),
        ],
    )(x, w)
    return loss[0, 0], perp[0, 0]


# speedup vs baseline: 2.1399x; 1.0130x over previous
import jax
import jax.numpy as jnp
from jax.experimental import pallas as pl
from jax.experimental.pallas import tpu as pltpu

DIM = 256
N_EMBED = 8192
COMMITMENT_COST = 0.25
TM = 256


def _vq_body(x_ref, w_ref, loss_ref, perp_ref, w2_ref, counts_ref, acc_ref):
    i = pl.program_id(0)
    ni = pl.num_programs(0)

    @pl.when(i == 0)
    def _init():
        w = w_ref[...].astype(jnp.float32)
        w2_ref[...] = jnp.sum(w * w, axis=0, keepdims=True)
        counts_ref[...] = jnp.zeros_like(counts_ref)
        acc_ref[...] = jnp.zeros_like(acc_ref)

    x = x_ref[...]
    xs = (-2.0 * x).astype(jnp.bfloat16)
    scores = jnp.dot(xs, w_ref[...], preferred_element_type=jnp.float32)
    scores = scores + w2_ref[...]
    m = jnp.min(scores, axis=1, keepdims=True)
    x2 = jnp.sum(x * x, axis=1, keepdims=True)
    acc_ref[...] += jnp.reshape(jnp.sum(m) + jnp.sum(x2), (1, 1))
    onehot = jnp.where(scores == m, 1.0, 0.0)
    counts_ref[...] += jnp.sum(onehot, axis=0, keepdims=True)

    @pl.when(i == ni - 1)
    def _fin():
        total = jnp.float32(TM) * jnp.float32(ni)
        p = counts_ref[...] / total
        ent = jnp.sum(p * jnp.log(p + 1e-10))
        perp_ref[...] = jnp.reshape(jnp.exp(-ent), (1, 1))
        loss_ref[...] = COMMITMENT_COST * acc_ref[...] / (total * DIM)


def kernel(input, w):
    x = input.reshape(-1, DIM)
    wb = w.astype(jnp.bfloat16)
    n = x.shape[0]
    ni = n // TM
    loss, perp = pl.pallas_call(
        _vq_body,
        grid=(ni,),
        in_specs=[
            pl.BlockSpec((TM, DIM), lambda i: (i, 0)),
            pl.BlockSpec((DIM, N_EMBED), lambda i: (0, 0)),
        ],
        out_specs=[
            pl.BlockSpec((1, 1), lambda i: (0, 0)),
            pl.BlockSpec((1, 1), lambda i: (0, 0)),
        ],
        out_shape=[
            jax.ShapeDtypeStruct((1, 1), jnp.float32),
            jax.ShapeDtypeStruct((1, 1), jnp.float32),
        ],
        scratch_shapes=[
            pltpu.VMEM((1, N_EMBED), jnp.float32),
            pltpu.VMEM((1, N_EMBED), jnp.float32),
            pltpu.VMEM((1, 1), jnp.float32),
        ],
    )(x, wb)
    return loss[0, 0], perp[0, 0]
